# Initial kernel scaffold; baseline (speedup 1.0000x reference)
import jax
import jax.numpy as jnp
from jax.experimental import pallas as pl


def kernel(node_features, demands, adj_lst, in_indices, num_nodes, dropout_keep_prob, emb_table, enc_W1, enc_b1, enc_W2, enc_b2, gat_W, gat_a_src, gat_a_dst, dec_W1, dec_b1, dec_W2, dec_b2, du_W1, du_b1, du_W2, du_b2):
    raise NotImplementedError("write your pallas kernel here")


# speedup vs baseline: 5.1089x; 5.1089x over previous
import functools

import jax
import jax.numpy as jnp
import numpy as np
from jax import lax
from jax.experimental import pallas as pl
from jax.experimental.pallas import tpu as pltpu
from jax.experimental.pallas import tpu_sc as plsc

N = 10000
K = 16
H = 4
ENC = 64
DH = ENC // H
BIG = 1.0e7
FLOW_ITERS = 8
DUAL_ITERS = 8
DUAL_STEP = 0.01
DUAL_MOM = 0.9

NPAD = 10240
NWORK = 32
CPW = NPAD // NWORK
CH = 64
NCH = CPW // CH
RB = 512
GRID = NPAD // RB

_MESH = plsc.VectorSubcoreMesh(core_axis_name="c", subcore_axis_name="s")
_MESH1 = plsc.VectorSubcoreMesh(
    core_axis_name="c", subcore_axis_name="s", num_cores=1)
NWORK_F = 16
CPW_F = NPAD // NWORK_F


def _wid():
    return lax.axis_index("s") * 2 + lax.axis_index("c")


def _enc_body(emb_ref, feat_ref, w1e_ref, w1f_ref, b1_ref, w2_ref, b2_ref,
              gw_ref, asrc_ref, adst_ref, h_ref, ssrc_ref, t_ref):
    emb = emb_ref[...]
    nrm = jnp.sqrt(jnp.sum(emb * emb, axis=1, keepdims=True))
    emb = emb / jnp.maximum(nrm, 1.0)
    x = jnp.tanh(jnp.dot(emb, w1e_ref[...], preferred_element_type=jnp.float32)
                 + jnp.dot(feat_ref[...], w1f_ref[...], preferred_element_type=jnp.float32)
                 + b1_ref[...])
    x = jnp.tanh(jnp.dot(x, w2_ref[...], preferred_element_type=jnp.float32) + b2_ref[...])
    h = jnp.dot(x, gw_ref[...], preferred_element_type=jnp.float32)
    h_ref[...] = h
    ssrc_ref[...] = jnp.dot(h, asrc_ref[...], preferred_element_type=jnp.float32)
    t_ref[...] = jnp.dot(h, adst_ref[...], preferred_element_type=jnp.float32)


def _mid_body(o_ref, gw_ref, asrc_ref, adst_ref, h_ref, ssrc_ref, t_ref):
    x = jnp.tanh(o_ref[...])
    h = jnp.dot(x, gw_ref[...], preferred_element_type=jnp.float32)
    h_ref[...] = h
    ssrc_ref[...] = jnp.dot(h, asrc_ref[...], preferred_element_type=jnp.float32)
    t_ref[...] = jnp.dot(h, adst_ref[...], preferred_element_type=jnp.float32)


def _post_body(o_ref, adj_ref, dm_ref, valid_ref, du1_ref, db1_ref, du2_ref,
               db2_ref, dv_ref, nw_ref, dem_ref, q_ref):
    x = jnp.tanh(o_ref[...])
    hdu = jnp.tanh(jnp.dot(x, du1_ref[...], preferred_element_type=jnp.float32) + db1_ref[...])
    dv_ref[...] = jnp.dot(hdu, du2_ref[...], preferred_element_type=jnp.float32) + db2_ref[...]
    maskf = (adj_ref[...] == N).astype(jnp.float32)
    one_m = 1.0 - maskf
    m = jnp.sum(one_m, axis=1, keepdims=True)
    nw = jnp.where(m > 0.0, one_m / jnp.maximum(m, 1.0), 1.0 / K) * valid_ref[...]
    nw_ref[...] = nw
    q_ref[...] = jnp.sum(nw * nw, axis=1, keepdims=True)
    dem_ref[...] = jnp.maximum(dm_ref[...], 0.0)


def _final_body(p_ref, o_ref):
    p = p_ref[...]
    val = jnp.sum(p[:, 0:1]) - jnp.sum(p[:, 1:2]) + jnp.sum(p[:, 2:3])
    o_ref[...] = val.reshape(1, 1)


def _full(shape):
    return pl.BlockSpec(shape, lambda i: (0, 0))


def _rows(width):
    return pl.BlockSpec((RB, width), lambda i: (i, 0))


def _gat_sc_body(h_hbm, adj_hbm, ssrc_hbm, t_hbm, out_hbm,
                 t_tab, adj_c, ssrc_c, icf, rows, out_c, sem):
    wid = _wid()
    pltpu.sync_copy(t_hbm, t_tab)
    zero16 = jnp.zeros((16,), jnp.float32)
    for c in range(NCH):
        row0 = wid * CPW + c * CH
        pltpu.sync_copy(adj_hbm.at[pl.ds(row0, CH)], adj_c)
        pltpu.sync_copy(ssrc_hbm.at[pl.ds(row0 * H, CH * H)], ssrc_c)

        def fill(g, _):
            a = adj_c[g, :]
            ic = jnp.minimum(a, N - 1)
            icf[g >> 3, pl.ds((g & 7) * K, K)] = ic
            return 0
        lax.fori_loop(0, CH, fill, 0)

        copies = []
        for i in range(CH * K // 128):
            copies.append(pltpu.async_copy(
                h_hbm.at[icf.at[i]], rows.at[pl.ds(i * 128, 128)], sem))
        for cp in copies:
            cp.wait()

        def node(n, _):
            a = adj_c[n, :]
            maskf = jnp.where(a == N, 1.0, 0.0)
            one_m = 1.0 - maskf
            ic = jnp.minimum(a, N - 1)
            icH = ic * H
            ws = []
            for h in range(H):
                tg = plsc.load_gather(t_tab, [icH + h])
                ss = plsc.load_gather(ssrc_c, [jnp.full((16,), n * H + h, jnp.int32)])
                z = ss + one_m * tg
                th = 1.0 - 2.0 / (jnp.exp(2.0 * z) + 1.0)
                sc = th - BIG * maskf
                p = jnp.exp(sc - jnp.max(sc))
                ws.append(p / jnp.sum(p) * one_m)
            accs = [zero16, zero16, zero16, zero16]
            for k in range(K):
                r = n * K + k
                for h in range(H):
                    accs[h] = accs[h] + ws[h][k] * rows[r, pl.ds(h * DH, DH)]
            for h in range(H):
                out_c[n, pl.ds(h * DH, DH)] = accs[h]
            return 0
        lax.fori_loop(0, CH, node, 0)
        pltpu.sync_copy(out_c, out_hbm.at[pl.ds(row0, CH)])


@functools.partial(
    pl.kernel, mesh=_MESH,
    compiler_params=pltpu.CompilerParams(
        needs_layout_passes=False, use_tc_tiling_on_sc=False),
    out_type=jax.ShapeDtypeStruct((NPAD, ENC), jnp.float32),
    scratch_types=[
        pltpu.VMEM((NPAD * H,), jnp.float32),
        pltpu.VMEM((CH, K), jnp.int32),
        pltpu.VMEM((CH * H,), jnp.float32),
        pltpu.VMEM((CH * K // 128, 128), jnp.int32),
        pltpu.VMEM((CH * K, ENC), jnp.float32),
        pltpu.VMEM((CH, ENC), jnp.float32),
        pltpu.SemaphoreType.DMA,
    ],
)
def _gat_sc(h_hbm, adj_hbm, ssrc_hbm, t_hbm, out_hbm, *scratch):
    _gat_sc_body(h_hbm, adj_hbm, ssrc_hbm, t_hbm, out_hbm, *scratch)


def _flow_sc_body(ii_hbm, adj_hbm, wf_hbm, dv_hbm, dem_hbm, dmraw_hbm, q_hbm,
                  part_hbm, ii_c, adj_c, wf_c, dvtab, stab, snew,
                  dem_c, dmraw_c, q_c, pbuf, shared, sem):
    wid = lax.axis_index("s")
    base = wid * CPW_F
    pltpu.sync_copy(ii_hbm.at[pl.ds(base * K, CPW_F * K)], ii_c)
    pltpu.sync_copy(adj_hbm.at[pl.ds(base * K, CPW_F * K)], adj_c)
    pltpu.sync_copy(dv_hbm, dvtab)
    pltpu.sync_copy(dem_hbm.at[pl.ds(base, CPW_F)], dem_c)
    pltpu.sync_copy(dmraw_hbm.at[pl.ds(base, CPW_F)], dmraw_c)
    pltpu.sync_copy(q_hbm.at[pl.ds(base, CPW_F)], q_c)

    copies = []
    for i in range(CPW_F * K // 128):
        copies.append(pltpu.async_copy(
            wf_hbm.at[ii_c.at[pl.ds(i * 128, 128)]],
            wf_c.at[pl.ds(i * 128, 128)], sem))
    for cp in copies:
        cp.wait()

    pltpu.sync_copy(dem_hbm, stab)
    plsc.subcore_barrier()

    def one_iter(it, _):
        def group(g, _g):
            nb = g * 16

            def lane(j, acc):
                n = nb + j
                iv = ii_c[pl.ds(n * K, K)]
                srcv = lax.shift_right_logical(iv, 4)
                sv = plsc.load_gather(stab, [srcv])
                wv = wf_c[pl.ds(n * K, K)]
                infl = jnp.sum(wv * sv)
                lanes = lax.iota(jnp.int32, 16)
                return jnp.where(lanes == j, infl, acc)
            acc = lax.fori_loop(0, 16, lane, jnp.zeros((16,), jnp.float32))
            snew[pl.ds(nb, 16)] = acc + dem_c[pl.ds(nb, 16)]
            return 0
        lax.fori_loop(0, CPW_F // 16, group, 0)
        pltpu.sync_copy(snew, shared.at[pl.ds(base, CPW_F)])
        plsc.subcore_barrier()
        pltpu.sync_copy(shared, stab)
        plsc.subcore_barrier()
        return 0
    lax.fori_loop(0, FLOW_ITERS - 1, one_iter, 0)

    def fc_group(g, acc):
        sl = pl.ds(g * 16, 16)
        sv = stab[pl.ds(base + g * 16, 16)]
        return acc + q_c[sl] * sv * sv
    fc_acc = lax.fori_loop(0, CPW_F // 16, fc_group, jnp.zeros((16,), jnp.float32))
    fc_part = jnp.sum(fc_acc)

    def dual_node(n, acc):
        a = adj_c[pl.ds(n * K, K)]
        maskf = jnp.where(a == N, 1.0, 0.0)
        one_m = 1.0 - maskf
        ic = jnp.minimum(a, N - 1)
        dg = plsc.load_gather(dvtab, [ic])
        dvn = plsc.load_gather(dvtab, [jnp.full((16,), 1, jnp.int32) * (base + n)])
        dd = one_m * dg - maskf * dvn
        df = jnp.zeros((16,), jnp.float32)
        mom = jnp.zeros((16,), jnp.float32)
        for _ in range(DUAL_ITERS):
            grad = 2.0 * df + dd
            mom = DUAL_MOM * mom - DUAL_STEP * grad
            df = jnp.maximum(df + mom, 0.0) * one_m
        return acc + df * df + dd * df
    dfc_acc = lax.fori_loop(0, CPW_F, dual_node, jnp.zeros((16,), jnp.float32))
    dfc_part = jnp.sum(dfc_acc)

    def dd_group(g, acc):
        sl = pl.ds(g * 16, 16)
        return acc + dvtab[pl.ds(base + g * 16, 16)] * dmraw_c[sl]
    dd_acc = lax.fori_loop(0, CPW_F // 16, dd_group, jnp.zeros((16,), jnp.float32))
    dd_part = jnp.sum(dd_acc)

    lanes = lax.iota(jnp.int32, 16)
    v = jnp.where(lanes == 0, fc_part,
                  jnp.where(lanes == 1, dfc_part,
                            jnp.where(lanes == 2, dd_part, 0.0)))
    pbuf[...] = v
    pltpu.sync_copy(pbuf, part_hbm.at[wid])


@functools.partial(
    pl.kernel, mesh=_MESH1,
    compiler_params=pltpu.CompilerParams(
        needs_layout_passes=False, use_tc_tiling_on_sc=False),
    out_type=jax.ShapeDtypeStruct((NWORK_F, 16), jnp.float32),
    scratch_types=[
        pltpu.VMEM((CPW_F * K,), jnp.int32),
        pltpu.VMEM((CPW_F * K,), jnp.int32),
        pltpu.VMEM((CPW_F * K,), jnp.float32),
        pltpu.VMEM((NPAD,), jnp.float32),
        pltpu.VMEM((NPAD,), jnp.float32),
        pltpu.VMEM((CPW_F,), jnp.float32),
        pltpu.VMEM((CPW_F,), jnp.float32),
        pltpu.VMEM((CPW_F,), jnp.float32),
        pltpu.VMEM((CPW_F,), jnp.float32),
        pltpu.VMEM((16,), jnp.float32),
        pltpu.VMEM_SHARED((NPAD,), jnp.float32),
        pltpu.SemaphoreType.DMA,
    ],
)
def _flow_sc(ii_hbm, adj_hbm, wf_hbm, dv_hbm, dem_hbm, dmraw_hbm, q_hbm,
             part_hbm, *scratch):
    _flow_sc_body(ii_hbm, adj_hbm, wf_hbm, dv_hbm, dem_hbm, dmraw_hbm, q_hbm,
                  part_hbm, *scratch)


def kernel(node_features, demands, adj_lst, in_indices, num_nodes,
           dropout_keep_prob, emb_table, enc_W1, enc_b1, enc_W2, enc_b2,
           gat_W, gat_a_src, gat_a_dst, dec_W1, dec_b1, dec_W2, dec_b2,
           du_W1, du_b1, du_W2, du_b2):
    pad = NPAD - N
    feat = jnp.pad(node_features[0], ((0, pad), (0, 0)))
    emb = jnp.pad(emb_table, ((0, pad), (0, 0)))
    adj = jnp.pad(adj_lst[0].astype(jnp.int32), ((0, pad), (0, 0)),
                  constant_values=N)
    ii = jnp.pad(in_indices[0].astype(jnp.int32), ((0, pad), (0, 0)))
    dm = jnp.pad(demands[0], ((0, pad), (0, 0)))

    eye = jnp.eye(H, dtype=jnp.float32)
    blk = jnp.repeat(eye, DH, axis=0).reshape(H, DH, H)
    a_src = (gat_a_src[:, :, None] * blk).reshape(ENC, H)
    a_dst = (gat_a_dst[:, :, None] * blk).reshape(ENC, H)

    enc_call = pl.pallas_call(
        _enc_body,
        grid=(GRID,),
        in_specs=[
            _rows(32), _rows(128),
            _full((32, ENC)), _full((128, ENC)),
            pl.BlockSpec((1, ENC), lambda i: (0, 0)),
            _full((ENC, ENC)), pl.BlockSpec((1, ENC), lambda i: (0, 0)),
            _full((ENC, ENC)), _full((ENC, H)), _full((ENC, H)),
        ],
        out_specs=[_rows(ENC), _rows(H), _rows(H)],
        out_shape=[
            jax.ShapeDtypeStruct((NPAD, ENC), jnp.float32),
            jax.ShapeDtypeStruct((NPAD, H), jnp.float32),
            jax.ShapeDtypeStruct((NPAD, H), jnp.float32),
        ],
    )
    h1, ssrc1, t1 = enc_call(
        emb, feat, enc_W1[:32], enc_W1[32:], enc_b1.reshape(1, ENC),
        enc_W2, enc_b2.reshape(1, ENC), gat_W, a_src, a_dst)

    out1 = _gat_sc(h1, adj, ssrc1.reshape(-1), t1.reshape(-1))

    mid_call = pl.pallas_call(
        _mid_body,
        grid=(GRID,),
        in_specs=[_rows(ENC), _full((ENC, ENC)), _full((ENC, H)), _full((ENC, H))],
        out_specs=[_rows(ENC), _rows(H), _rows(H)],
        out_shape=[
            jax.ShapeDtypeStruct((NPAD, ENC), jnp.float32),
            jax.ShapeDtypeStruct((NPAD, H), jnp.float32),
            jax.ShapeDtypeStruct((NPAD, H), jnp.float32),
        ],
    )
    h2, ssrc2, t2 = mid_call(out1, gat_W, a_src, a_dst)

    out2 = _gat_sc(h2, adj, ssrc2.reshape(-1), t2.reshape(-1))

    post_call = pl.pallas_call(
        _post_body,
        grid=(GRID,),
        in_specs=[
            _rows(ENC), _rows(K), _rows(1), _rows(1),
            _full((ENC, 32)), pl.BlockSpec((1, 32), lambda i: (0, 0)),
            _full((32, 1)), pl.BlockSpec((1, 1), lambda i: (0, 0)),
        ],
        out_specs=[_rows(1), _rows(K), _rows(1), _rows(1)],
        out_shape=[
            jax.ShapeDtypeStruct((NPAD, 1), jnp.float32),
            jax.ShapeDtypeStruct((NPAD, K), jnp.float32),
            jax.ShapeDtypeStruct((NPAD, 1), jnp.float32),
            jax.ShapeDtypeStruct((NPAD, 1), jnp.float32),
        ],
    )
    valid = jnp.pad(jnp.ones((N, 1), jnp.float32), ((0, pad), (0, 0)))
    dv, nw, dem, q = post_call(
        out2, adj, dm, valid, du_W1, du_b1.reshape(1, 32),
        du_W2, du_b2.reshape(1, 1))

    parts = _flow_sc(
        ii.reshape(-1), adj.reshape(-1), nw.reshape(-1), dv.reshape(-1),
        dem.reshape(-1), dm.reshape(-1), q.reshape(-1))

    final_call = pl.pallas_call(
        _final_body,
        in_specs=[pl.BlockSpec((NWORK_F, 16), lambda: (0, 0))],
        out_specs=pl.BlockSpec((1, 1), lambda: (0, 0)),
        out_shape=jax.ShapeDtypeStruct((1, 1), jnp.float32),
    )
    loss = final_call(parts)
    return loss[0, 0]
